# Initial kernel scaffold; baseline (speedup 1.0000x reference)
import jax
import jax.numpy as jnp
from jax.experimental import pallas as pl


def kernel(x, edge_index, edge_attr, W1, b1, W2, b2):
    raise NotImplementedError("write your pallas kernel here")


# speedup vs baseline: 8.3515x; 8.3515x over previous
import functools

import jax
import jax.numpy as jnp
from jax import lax
from jax.experimental import pallas as pl
from jax.experimental.pallas import tpu as pltpu
from jax.experimental.pallas import tpu_sc as plsc

N = 10000
NP = 10240
E = 320000
D = 128
F = D // 2
PACK = 16384

NC = 2
NS = 16
EPT = E // NS
CH = 80
NCHUNK = EPT // CH
RPT = NP // NS
ZR = 8
ZF = 128
BLK = 1024

_SC_PARAMS = pltpu.CompilerParams(use_tc_tiling_on_sc=False)


def _linear_body(x_ref, w_ref, b_ref, y_ref):
    y_ref[...] = lax.dot_general(
        x_ref[...], w_ref[...], (((1,), (1,)), ((), ())),
        preferred_element_type=jnp.float32) + b_ref[0]


def _linear(x, w, b3):
    return pl.pallas_call(
        _linear_body,
        grid=(NP // BLK, NC),
        in_specs=[
            pl.BlockSpec((BLK, D), lambda i, k: (i, 0)),
            pl.BlockSpec((F, D), lambda i, k: (k, 0)),
            pl.BlockSpec((1, 1, F), lambda i, k: (k, 0, 0)),
        ],
        out_specs=pl.BlockSpec((BLK, F), lambda i, k: (k * (NP // BLK) + i, 0)),
        out_shape=jax.ShapeDtypeStruct((NC * NP, F), jnp.float32),
    )(x, w, b3)


def _combine_linear_body(a0_ref, a1_ref, c_ref, w_ref, b_ref, y_ref, r_ref):
    r = 1.0 / jnp.maximum(c_ref[...], 1.0)
    h = jnp.maximum(
        jnp.concatenate([a0_ref[...], a1_ref[...]], axis=1) * r, 0.0)
    y_ref[...] = lax.dot_general(
        h, w_ref[...], (((1,), (1,)), ((), ())),
        preferred_element_type=jnp.float32) + b_ref[0]
    r_ref[...] = r


def _combine_linear(acc, cnt, w, b3):
    return pl.pallas_call(
        _combine_linear_body,
        grid=(NP // BLK, NC),
        in_specs=[
            pl.BlockSpec((BLK, F), lambda i, k: (i, 0)),
            pl.BlockSpec((BLK, F), lambda i, k: (NP // BLK + i, 0)),
            pl.BlockSpec((BLK, 1), lambda i, k: (i, 0)),
            pl.BlockSpec((F, D), lambda i, k: (k, 0)),
            pl.BlockSpec((1, 1, F), lambda i, k: (k, 0, 0)),
        ],
        out_specs=[
            pl.BlockSpec((BLK, F), lambda i, k: (k * (NP // BLK) + i, 0)),
            pl.BlockSpec((BLK, 1), lambda i, k: (i, 0)),
        ],
        out_shape=[
            jax.ShapeDtypeStruct((NC * NP, F), jnp.float32),
            jax.ShapeDtypeStruct((NP, 1), jnp.float32),
        ],
    )(acc, acc, cnt, w, b3)


def _finalize_body(a0_ref, a1_ref, r_ref, o_ref):
    o_ref[...] = jnp.maximum(
        jnp.concatenate([a0_ref[...], a1_ref[...]], axis=1) * r_ref[...], 0.0)


def _finalize(acc, r):
    return pl.pallas_call(
        _finalize_body,
        grid=(NP // BLK,),
        in_specs=[
            pl.BlockSpec((BLK, F), lambda i: (i, 0)),
            pl.BlockSpec((BLK, F), lambda i: (NP // BLK + i, 0)),
            pl.BlockSpec((BLK, 1), lambda i: (i, 0)),
        ],
        out_specs=pl.BlockSpec((BLK, D), lambda i: (i, 0)),
        out_shape=jax.ShapeDtypeStruct((NP, D), jnp.float32),
    )(acc, acc, r)


def _sc_layer_body(with_count, *refs):
    if with_count:
        (y_hbm, sd_hbm, ea_hbm, wcol_hbm, acc_out, cnt_out,
         pkb, eab, srcb, dstb, rows, wcol_v, zrow_v, zflat_v, ones_v,
         acc_sh, cnt_sh,
         ssem0, ssem1, gsem0, gsem1, wsem0, wsem1, csem0, csem1) = refs
        csem = (csem0, csem1)
    else:
        (y_hbm, sd_hbm, ea_hbm, wcol_hbm, acc_out,
         pkb, eab, srcb, dstb, rows, wcol_v, zrow_v,
         acc_sh,
         ssem0, ssem1, gsem0, gsem1, wsem0, wsem1) = refs
    ssem = (ssem0, ssem1)
    gsem = (gsem0, gsem1)
    wsem = (wsem0, wsem1)

    cid = lax.axis_index("c")
    sid = lax.axis_index("s")

    zeros16 = jnp.zeros((16,), jnp.float32)

    for i in range(ZR):
        for v in range(F // 16):
            zrow_v[i, pl.ds(v * 16, 16)] = zeros16

    def _zacc(k, c):
        pltpu.sync_copy(zrow_v, acc_sh.at[pl.ds(sid * RPT + k * ZR, ZR)])
        return c
    lax.fori_loop(0, RPT // ZR, _zacc, 0)

    if with_count:
        @pl.when(jnp.logical_and(cid == 0, sid == 0))
        def _zero_cnt():
            for i in range(ZF // 16):
                zflat_v[pl.ds(i * 16, 16)] = zeros16

            def _zc(k, c):
                pltpu.sync_copy(zflat_v, cnt_sh.at[pl.ds(k * ZF, ZF)])
                return c
            lax.fori_loop(0, NP // ZF, _zc, 0)
        for t in range(CH // 16):
            ones_v[pl.ds(t * 16, 16)] = jnp.full((16,), 1.0, jnp.float32)

    pltpu.sync_copy(wcol_hbm.at[cid], wcol_v)
    wc = [wcol_v[pl.ds(v * 16, 16)] for v in range(F // 16)]
    dmask = jnp.full((16,), PACK - 1, jnp.int32)
    ybase = jnp.full((16,), cid * NP, jnp.int32)

    plsc.subcore_barrier()

    def _stage(j, b):
        pltpu.async_copy(sd_hbm.at[sid, pl.ds(j * CH, CH)], pkb.at[b],
                         ssem[b])
        pltpu.async_copy(ea_hbm.at[sid, pl.ds(j * CH, CH)], eab.at[b],
                         ssem[b])

    def _stage_wait(b):
        pltpu.make_async_copy(sd_hbm.at[sid, pl.ds(0, CH)], pkb.at[b],
                              ssem[b]).wait()
        pltpu.make_async_copy(ea_hbm.at[sid, pl.ds(0, CH)], eab.at[b],
                              ssem[b]).wait()

    def _unpack(b):
        for k in range(CH // 16):
            sl = pl.ds(k * 16, 16)
            p = pkb[b, sl]
            srcb[b, sl] = lax.shift_right_logical(p, 14) + ybase
            dstb[b, sl] = lax.bitwise_and(p, dmask)

    def _gather(b):
        pltpu.async_copy(y_hbm.at[srcb.at[b]], rows.at[b], gsem[b])

    def _gather_wait(b):
        pltpu.make_async_copy(y_hbm.at[srcb.at[b]], rows.at[b],
                              gsem[b]).wait()

    def _scatter(b):
        pltpu.async_copy(rows.at[b], acc_sh.at[dstb.at[b]], wsem[b],
                         add=True)
        if with_count:
            @pl.when(cid == 0)
            def _():
                pltpu.async_copy(ones_v, cnt_sh.at[dstb.at[b]], csem[b],
                                 add=True)

    def _scatter_wait(b):
        pltpu.make_async_copy(rows.at[b], acc_sh.at[dstb.at[b]],
                              wsem[b]).wait()
        if with_count:
            @pl.when(cid == 0)
            def _():
                pltpu.make_async_copy(ones_v, cnt_sh.at[dstb.at[b]],
                                      csem[b]).wait()

    def _compute(b, ea16s):
        for g in range(CH // 16):
            for k in range(16):
                e = g * 16 + k
                eav = jnp.full((16,), ea16s[g][k], jnp.float32)
                for v in range(F // 16):
                    sl = pl.ds(v * 16, 16)
                    rows[b, e, sl] = jnp.maximum(
                        rows[b, e, sl] + eav * wc[v], 0.0)

    _stage(0, 0)
    _stage(1, 1)
    _stage_wait(0)
    _unpack(0)
    _gather(0)

    def _pair(p, c):
        for b in range(2):
            j = 2 * p + b
            nb = 1 - b

            @pl.when(j + 1 < NCHUNK)
            def _prep_next():
                _stage_wait(nb)

                @pl.when(j >= 1)
                def _():
                    _scatter_wait(nb)
                _unpack(nb)
                _gather(nb)

            ea16s = [eab[b, pl.ds(g * 16, 16)] for g in range(CH // 16)]

            @pl.when(j + 2 < NCHUNK)
            def _stage_ahead():
                _stage(j + 2, b)
            _gather_wait(b)
            _compute(b, ea16s)
            _scatter(b)
        return c
    lax.fori_loop(0, NCHUNK // 2, _pair, 0)

    _scatter_wait(0)
    _scatter_wait(1)

    plsc.subcore_barrier()

    pltpu.sync_copy(acc_sh.at[pl.ds(sid * RPT, RPT)],
                    acc_out.at[pl.ds(cid * NP + sid * RPT, RPT)])
    if with_count:
        @pl.when(jnp.logical_and(cid == 0, sid == 0))
        def _copy_cnt():
            pltpu.sync_copy(cnt_sh, cnt_out)


def _make_sc_layer(with_count):
    mesh = plsc.VectorSubcoreMesh(
        core_axis_name="c", subcore_axis_name="s",
        num_cores=NC, num_subcores=NS)
    out_type = [jax.ShapeDtypeStruct((NC * NP, F), jnp.float32)]
    scratch = [
        pltpu.VMEM((2, CH), jnp.int32),
        pltpu.VMEM((2, CH), jnp.float32),
        pltpu.VMEM((2, CH), jnp.int32),
        pltpu.VMEM((2, CH), jnp.int32),
        pltpu.VMEM((2, CH, F), jnp.float32),
        pltpu.VMEM((F,), jnp.float32),
        pltpu.VMEM((ZR, F), jnp.float32),
    ]
    if with_count:
        out_type.append(jax.ShapeDtypeStruct((NP,), jnp.float32))
        scratch += [
            pltpu.VMEM((ZF,), jnp.float32),
            pltpu.VMEM((CH,), jnp.float32),
        ]
    scratch += [pltpu.VMEM_SHARED((NP, F), jnp.float32)]
    if with_count:
        scratch += [pltpu.VMEM_SHARED((NP,), jnp.float32)]
    nsem = 8 if with_count else 6
    scratch += [pltpu.SemaphoreType.DMA] * nsem
    return pl.kernel(
        functools.partial(_sc_layer_body, with_count),
        out_type=tuple(out_type),
        mesh=mesh,
        scratch_types=scratch,
        compiler_params=_SC_PARAMS,
    )


_sc_layer_with_count = _make_sc_layer(True)
_sc_layer_no_count = _make_sc_layer(False)


def kernel(x, edge_index, edge_attr, W1, b1, W2, b2):
    ei = edge_index.astype(jnp.int32)
    sd = (ei[0] * PACK + ei[1]).reshape(NS, EPT)
    ea = edge_attr.reshape(NS, EPT).astype(jnp.float32)

    w1k, wcol1 = W1[:, :D], W1[:, D].reshape(NC, F)
    w2k, wcol2 = W2[:, :D], W2[:, D].reshape(NC, F)
    b1r = b1.reshape(NC, 1, F)
    b2r = b2.reshape(NC, 1, F)

    xp = jnp.pad(x.astype(jnp.float32), ((0, NP - N), (0, 0)))
    y1 = _linear(xp, w1k, b1r)
    acc1, cnt = _sc_layer_with_count(y1, sd, ea, wcol1)
    y2, r = _combine_linear(acc1, cnt.reshape(NP, 1), w2k, b2r)
    acc2, = _sc_layer_no_count(y2, sd, ea, wcol2)
    return _finalize(acc2, r)[:N]


# speedup vs baseline: 8.9492x; 1.0716x over previous
import functools

import jax
import jax.numpy as jnp
from jax import lax
from jax.experimental import pallas as pl
from jax.experimental.pallas import tpu as pltpu
from jax.experimental.pallas import tpu_sc as plsc

N = 10000
NP = 10240
E = 320000
D = 128
F = D // 2

NC = 2
NS = 16
EPT = E // NS
CH = 80
NCHUNK = EPT // CH
RPT = NP // NS
ZR = 8
ZF = 128

_SC_PARAMS = pltpu.CompilerParams(use_tc_tiling_on_sc=False)


def _linear_body(x_ref, w_ref, b_ref, y_ref):
    y_ref[0:N, :] = lax.dot_general(
        x_ref[...], w_ref[...], (((1,), (1,)), ((), ())),
        preferred_element_type=jnp.float32) + b_ref[0]


def _linear(x, w, b3):
    return pl.pallas_call(
        _linear_body,
        grid=(NC,),
        in_specs=[
            pl.BlockSpec((N, D), lambda k: (0, 0)),
            pl.BlockSpec((F, D), lambda k: (k, 0)),
            pl.BlockSpec((1, 1, F), lambda k: (k, 0, 0)),
        ],
        out_specs=pl.BlockSpec((NP, F), lambda k: (k, 0)),
        out_shape=jax.ShapeDtypeStruct((NC * NP, F), jnp.float32),
    )(x, w, b3)


def _combine_linear_body(a0_ref, a1_ref, c_ref, w_ref, b_ref, y_ref, r_ref):
    r = 1.0 / jnp.maximum(c_ref[...], 1.0)
    h = jnp.maximum(
        jnp.concatenate([a0_ref[...], a1_ref[...]], axis=1) * r, 0.0)
    y_ref[...] = lax.dot_general(
        h, w_ref[...], (((1,), (1,)), ((), ())),
        preferred_element_type=jnp.float32) + b_ref[0]
    r_ref[...] = r


def _combine_linear(acc, cnt, w, b3):
    return pl.pallas_call(
        _combine_linear_body,
        grid=(NC,),
        in_specs=[
            pl.BlockSpec((NP, F), lambda k: (0, 0)),
            pl.BlockSpec((NP, F), lambda k: (1, 0)),
            pl.BlockSpec((NP, 1), lambda k: (0, 0)),
            pl.BlockSpec((F, D), lambda k: (k, 0)),
            pl.BlockSpec((1, 1, F), lambda k: (k, 0, 0)),
        ],
        out_specs=[
            pl.BlockSpec((NP, F), lambda k: (k, 0)),
            pl.BlockSpec((NP, 1), lambda k: (0, 0)),
        ],
        out_shape=[
            jax.ShapeDtypeStruct((NC * NP, F), jnp.float32),
            jax.ShapeDtypeStruct((NP, 1), jnp.float32),
        ],
    )(acc, acc, cnt, w, b3)


def _finalize_body(a0_ref, a1_ref, r_ref, o_ref):
    o_ref[...] = jnp.maximum(
        jnp.concatenate([a0_ref[0:N, :], a1_ref[0:N, :]], axis=1)
        * r_ref[0:N, :], 0.0)


def _finalize(acc, r):
    return pl.pallas_call(
        _finalize_body,
        grid=(1,),
        in_specs=[
            pl.BlockSpec((NP, F), lambda k: (0, 0)),
            pl.BlockSpec((NP, F), lambda k: (1, 0)),
            pl.BlockSpec((NP, 1), lambda k: (0, 0)),
        ],
        out_specs=pl.BlockSpec((N, D), lambda k: (0, 0)),
        out_shape=jax.ShapeDtypeStruct((N, D), jnp.float32),
    )(acc, acc, r)


def _sc_layer_body(with_count, *refs):
    if with_count:
        (y_hbm, src_hbm, dst_hbm, ea_hbm, wcol_hbm, acc_out, cnt_out,
         skb, dkb, eab, srcb, dstb, rows, wcol_v, zrow_v, zflat_v, ones_v,
         acc_sh, cnt_sh,
         ssem0, ssem1, gsem0, gsem1, wsem0, wsem1, csem0, csem1) = refs
        csem = (csem0, csem1)
    else:
        (y_hbm, src_hbm, dst_hbm, ea_hbm, wcol_hbm, acc_out,
         skb, dkb, eab, srcb, dstb, rows, wcol_v, zrow_v,
         acc_sh,
         ssem0, ssem1, gsem0, gsem1, wsem0, wsem1) = refs
    ssem = (ssem0, ssem1)
    gsem = (gsem0, gsem1)
    wsem = (wsem0, wsem1)

    cid = lax.axis_index("c")
    sid = lax.axis_index("s")

    zeros16 = jnp.zeros((16,), jnp.float32)

    for i in range(ZR):
        for v in range(F // 16):
            zrow_v[i, pl.ds(v * 16, 16)] = zeros16

    def _zacc(k, c):
        pltpu.sync_copy(zrow_v, acc_sh.at[pl.ds(sid * RPT + k * ZR, ZR)])
        return c
    lax.fori_loop(0, RPT // ZR, _zacc, 0)

    if with_count:
        @pl.when(jnp.logical_and(cid == 0, sid == 0))
        def _zero_cnt():
            for i in range(ZF // 16):
                zflat_v[pl.ds(i * 16, 16)] = zeros16

            def _zc(k, c):
                pltpu.sync_copy(zflat_v, cnt_sh.at[pl.ds(k * ZF, ZF)])
                return c
            lax.fori_loop(0, NP // ZF, _zc, 0)
        for t in range(CH // 16):
            ones_v[pl.ds(t * 16, 16)] = jnp.full((16,), 1.0, jnp.float32)

    pltpu.sync_copy(wcol_hbm.at[cid], wcol_v)
    wc = [wcol_v[pl.ds(v * 16, 16)] for v in range(F // 16)]
    ybase = jnp.full((16,), cid * NP, jnp.int32)

    plsc.subcore_barrier()

    def _stage(j, b):
        sl = pl.ds(j * CH, CH)
        pltpu.async_copy(src_hbm.at[sid, sl], skb.at[b], ssem[b])
        pltpu.async_copy(dst_hbm.at[sid, sl], dkb.at[b], ssem[b])
        pltpu.async_copy(ea_hbm.at[sid, sl], eab.at[b], ssem[b])

    def _stage_wait(b):
        sl = pl.ds(0, CH)
        pltpu.make_async_copy(src_hbm.at[sid, sl], skb.at[b], ssem[b]).wait()
        pltpu.make_async_copy(dst_hbm.at[sid, sl], dkb.at[b], ssem[b]).wait()
        pltpu.make_async_copy(ea_hbm.at[sid, sl], eab.at[b], ssem[b]).wait()

    def _unpack(b):
        for k in range(CH // 16):
            sl = pl.ds(k * 16, 16)
            srcb[b, sl] = skb[b, sl] + ybase
            dstb[b, sl] = dkb[b, sl]

    def _gather(b):
        pltpu.async_copy(y_hbm.at[srcb.at[b]], rows.at[b], gsem[b])

    def _gather_wait(b):
        pltpu.make_async_copy(y_hbm.at[srcb.at[b]], rows.at[b],
                              gsem[b]).wait()

    def _scatter(b):
        pltpu.async_copy(rows.at[b], acc_sh.at[dstb.at[b]], wsem[b],
                         add=True)
        if with_count:
            @pl.when(cid == 0)
            def _():
                pltpu.async_copy(ones_v, cnt_sh.at[dstb.at[b]], csem[b],
                                 add=True)

    def _scatter_wait(b):
        pltpu.make_async_copy(rows.at[b], acc_sh.at[dstb.at[b]],
                              wsem[b]).wait()
        if with_count:
            @pl.when(cid == 0)
            def _():
                pltpu.make_async_copy(ones_v, cnt_sh.at[dstb.at[b]],
                                      csem[b]).wait()

    def _compute(b, ea16s):
        for g in range(CH // 16):
            for k in range(16):
                e = g * 16 + k
                eav = jnp.full((16,), ea16s[g][k], jnp.float32)
                for v in range(F // 16):
                    sl = pl.ds(v * 16, 16)
                    rows[b, e, sl] = jnp.maximum(
                        rows[b, e, sl] + eav * wc[v], 0.0)

    _stage(0, 0)
    _stage(1, 1)
    _stage_wait(0)
    _unpack(0)
    _gather(0)

    def _pair(p, c):
        for b in range(2):
            j = 2 * p + b
            nb = 1 - b

            @pl.when(j + 1 < NCHUNK)
            def _prep_next():
                _stage_wait(nb)

                @pl.when(j >= 1)
                def _():
                    _scatter_wait(nb)
                _unpack(nb)
                _gather(nb)

            ea16s = [eab[b, pl.ds(g * 16, 16)] for g in range(CH // 16)]

            @pl.when(j + 2 < NCHUNK)
            def _stage_ahead():
                _stage(j + 2, b)
            _gather_wait(b)
            _compute(b, ea16s)
            _scatter(b)
        return c
    lax.fori_loop(0, NCHUNK // 2, _pair, 0)

    _scatter_wait(0)
    _scatter_wait(1)

    plsc.subcore_barrier()

    pltpu.sync_copy(acc_sh.at[pl.ds(sid * RPT, RPT)],
                    acc_out.at[pl.ds(cid * NP + sid * RPT, RPT)])
    if with_count:
        @pl.when(jnp.logical_and(cid == 0, sid == 0))
        def _copy_cnt():
            pltpu.sync_copy(cnt_sh, cnt_out)


def _make_sc_layer(with_count):
    mesh = plsc.VectorSubcoreMesh(
        core_axis_name="c", subcore_axis_name="s",
        num_cores=NC, num_subcores=NS)
    out_type = [jax.ShapeDtypeStruct((NC * NP, F), jnp.float32)]
    scratch = [
        pltpu.VMEM((2, CH), jnp.int32),
        pltpu.VMEM((2, CH), jnp.int32),
        pltpu.VMEM((2, CH), jnp.float32),
        pltpu.VMEM((2, CH), jnp.int32),
        pltpu.VMEM((2, CH), jnp.int32),
        pltpu.VMEM((2, CH, F), jnp.float32),
        pltpu.VMEM((F,), jnp.float32),
        pltpu.VMEM((ZR, F), jnp.float32),
    ]
    if with_count:
        out_type.append(jax.ShapeDtypeStruct((NP,), jnp.float32))
        scratch += [
            pltpu.VMEM((ZF,), jnp.float32),
            pltpu.VMEM((CH,), jnp.float32),
        ]
    scratch += [pltpu.VMEM_SHARED((NP, F), jnp.float32)]
    if with_count:
        scratch += [pltpu.VMEM_SHARED((NP,), jnp.float32)]
    nsem = 8 if with_count else 6
    scratch += [pltpu.SemaphoreType.DMA] * nsem
    return pl.kernel(
        functools.partial(_sc_layer_body, with_count),
        out_type=tuple(out_type),
        mesh=mesh,
        scratch_types=scratch,
        compiler_params=_SC_PARAMS,
    )


_sc_layer_with_count = _make_sc_layer(True)
_sc_layer_no_count = _make_sc_layer(False)


def kernel(x, edge_index, edge_attr, W1, b1, W2, b2):
    ei = edge_index.astype(jnp.int32)
    src = ei[0].reshape(NS, EPT)
    dst = ei[1].reshape(NS, EPT)
    ea = edge_attr.reshape(NS, EPT).astype(jnp.float32)

    w1k, wcol1 = W1[:, :D], W1[:, D].reshape(NC, F)
    w2k, wcol2 = W2[:, :D], W2[:, D].reshape(NC, F)
    b1r = b1.reshape(NC, 1, F)
    b2r = b2.reshape(NC, 1, F)

    y1 = _linear(x.astype(jnp.float32), w1k, b1r)
    acc1, cnt = _sc_layer_with_count(y1, src, dst, ea, wcol1)
    y2, r = _combine_linear(acc1, cnt.reshape(NP, 1), w2k, b2r)
    acc2, = _sc_layer_no_count(y2, src, dst, ea, wcol2)
    return _finalize(acc2, r)


# speedup vs baseline: 9.5454x; 1.0666x over previous
import functools

import jax
import jax.numpy as jnp
from jax import lax
from jax.experimental import pallas as pl
from jax.experimental.pallas import tpu as pltpu
from jax.experimental.pallas import tpu_sc as plsc

N = 10000
NP = 10240
E = 320000
D = 128
F = D // 2

NC = 2
NS = 16
EPT = E // NS
CH = 80
NCHUNK = EPT // CH
BPC = 10
SBLK = BPC * CH
NBLK = NCHUNK // BPC
RPT = NP // NS
ZR = 8
ZF = 128

_SC_PARAMS = pltpu.CompilerParams(use_tc_tiling_on_sc=False)


def _linear_body(x_ref, w_ref, b_ref, ei_ref, y_ref, src_ref, dst_ref):
    y_ref[0:N, :] = lax.dot_general(
        x_ref[...], w_ref[...], (((1,), (1,)), ((), ())),
        preferred_element_type=jnp.float32) + b_ref[0]

    @pl.when(pl.program_id(0) == 0)
    def _split_edges():
        src_ref[...] = ei_ref[0, :]
        dst_ref[...] = ei_ref[1, :]


def _linear(x, w, b3, ei):
    return pl.pallas_call(
        _linear_body,
        grid=(NC,),
        in_specs=[
            pl.BlockSpec((N, D), lambda k: (0, 0)),
            pl.BlockSpec((F, D), lambda k: (k, 0)),
            pl.BlockSpec((1, 1, F), lambda k: (k, 0, 0)),
            pl.BlockSpec((2, E), lambda k: (0, 0)),
        ],
        out_specs=[
            pl.BlockSpec((NP, F), lambda k: (k, 0)),
            pl.BlockSpec((E,), lambda k: (0,)),
            pl.BlockSpec((E,), lambda k: (0,)),
        ],
        out_shape=[
            jax.ShapeDtypeStruct((NC * NP, F), jnp.float32),
            jax.ShapeDtypeStruct((E,), jnp.int32),
            jax.ShapeDtypeStruct((E,), jnp.int32),
        ],
    )(x, w, b3, ei)


def _combine_linear_body(a_ref, c_ref, w_ref, b_ref, y_ref, r_ref):
    r = 1.0 / jnp.maximum(c_ref[...], 1.0)
    h = jnp.maximum(a_ref[...] * r, 0.0)
    y_ref[...] = lax.dot_general(
        h, w_ref[...], (((1,), (1,)), ((), ())),
        preferred_element_type=jnp.float32) + b_ref[0]
    r_ref[...] = r


def _combine_linear(acc, cnt, w, b3):
    return pl.pallas_call(
        _combine_linear_body,
        grid=(NC,),
        in_specs=[
            pl.BlockSpec((NP, D), lambda k: (0, 0)),
            pl.BlockSpec((NP, 1), lambda k: (0, 0)),
            pl.BlockSpec((F, D), lambda k: (k, 0)),
            pl.BlockSpec((1, 1, F), lambda k: (k, 0, 0)),
        ],
        out_specs=[
            pl.BlockSpec((NP, F), lambda k: (k, 0)),
            pl.BlockSpec((NP, 1), lambda k: (0, 0)),
        ],
        out_shape=[
            jax.ShapeDtypeStruct((NC * NP, F), jnp.float32),
            jax.ShapeDtypeStruct((NP, 1), jnp.float32),
        ],
    )(acc, cnt, w, b3)


def _finalize_body(a_ref, r_ref, o_ref):
    o_ref[...] = jnp.maximum(a_ref[0:N, :] * r_ref[0:N, :], 0.0)


def _finalize(acc, r):
    return pl.pallas_call(
        _finalize_body,
        grid=(1,),
        in_specs=[
            pl.BlockSpec((NP, D), lambda k: (0, 0)),
            pl.BlockSpec((NP, 1), lambda k: (0, 0)),
        ],
        out_specs=pl.BlockSpec((N, D), lambda k: (0, 0)),
        out_shape=jax.ShapeDtypeStruct((N, D), jnp.float32),
    )(acc, r)


def _sc_layer_body(with_count, *refs):
    if with_count:
        (y_hbm, src_hbm, dst_hbm, ea_hbm, wcol_hbm, acc_out, cnt_out,
         skb, dkb, eab, srcb, dstb, rows, wcol_v, zrow_v, zflat_v, ones_v,
         acc_sh, cnt_sh,
         ssem0, ssem1, gsem0, gsem1, wsem0, wsem1, csem0, csem1) = refs
        csem = (csem0, csem1)
    else:
        (y_hbm, src_hbm, dst_hbm, ea_hbm, wcol_hbm, acc_out,
         skb, dkb, eab, srcb, dstb, rows, wcol_v, zrow_v,
         acc_sh,
         ssem0, ssem1, gsem0, gsem1, wsem0, wsem1) = refs
    ssem = (ssem0, ssem1)
    gsem = (gsem0, gsem1)
    wsem = (wsem0, wsem1)

    cid = lax.axis_index("c")
    sid = lax.axis_index("s")

    zeros16 = jnp.zeros((16,), jnp.float32)

    for i in range(ZR):
        for v in range(F // 16):
            zrow_v[i, pl.ds(v * 16, 16)] = zeros16

    def _zacc(k, c):
        pltpu.sync_copy(zrow_v, acc_sh.at[pl.ds(sid * RPT + k * ZR, ZR)])
        return c
    lax.fori_loop(0, RPT // ZR, _zacc, 0)

    if with_count:
        @pl.when(jnp.logical_and(cid == 0, sid == 0))
        def _zero_cnt():
            for i in range(ZF // 16):
                zflat_v[pl.ds(i * 16, 16)] = zeros16

            def _zc(k, c):
                pltpu.sync_copy(zflat_v, cnt_sh.at[pl.ds(k * ZF, ZF)])
                return c
            lax.fori_loop(0, NP // ZF, _zc, 0)
        for t in range(CH // 16):
            ones_v[pl.ds(t * 16, 16)] = jnp.full((16,), 1.0, jnp.float32)

    pltpu.sync_copy(wcol_hbm.at[cid], wcol_v)
    wc = [wcol_v[pl.ds(v * 16, 16)] for v in range(F // 16)]
    ybase = jnp.full((16,), cid * NP, jnp.int32)

    plsc.subcore_barrier()

    ebase = sid * EPT

    def _stage(blk, sb):
        sl = pl.ds(ebase + blk * SBLK, SBLK)
        pltpu.async_copy(src_hbm.at[sl], skb.at[sb], ssem[sb])
        pltpu.async_copy(dst_hbm.at[sl], dkb.at[sb], ssem[sb])
        pltpu.async_copy(ea_hbm.at[sl], eab.at[sb], ssem[sb])

    def _stage_wait(sb):
        sl = pl.ds(0, SBLK)
        pltpu.make_async_copy(src_hbm.at[sl], skb.at[sb], ssem[sb]).wait()
        pltpu.make_async_copy(dst_hbm.at[sl], dkb.at[sb], ssem[sb]).wait()
        pltpu.make_async_copy(ea_hbm.at[sl], eab.at[sb], ssem[sb]).wait()

    def _unpack(c, b):
        sb = lax.rem(lax.div(c, BPC), 2)
        off = lax.rem(c, BPC) * CH
        for k in range(CH // 16):
            sl = pl.ds(k * 16, 16)
            bsl = pl.ds(off + k * 16, 16)
            srcb[b, sl] = skb[sb, bsl] + ybase
            dstb[b, sl] = dkb[sb, bsl]

    def _gather(b):
        pltpu.async_copy(y_hbm.at[srcb.at[b]], rows.at[b], gsem[b])

    def _gather_wait(b):
        pltpu.make_async_copy(y_hbm.at[srcb.at[b]], rows.at[b],
                              gsem[b]).wait()

    def _scatter(b):
        pltpu.async_copy(rows.at[b], acc_sh.at[dstb.at[b]], wsem[b],
                         add=True)
        if with_count:
            @pl.when(cid == 0)
            def _():
                pltpu.async_copy(ones_v, cnt_sh.at[dstb.at[b]], csem[b],
                                 add=True)

    def _scatter_wait(b):
        pltpu.make_async_copy(rows.at[b], acc_sh.at[dstb.at[b]],
                              wsem[b]).wait()
        if with_count:
            @pl.when(cid == 0)
            def _():
                pltpu.make_async_copy(ones_v, cnt_sh.at[dstb.at[b]],
                                      csem[b]).wait()

    def _compute(b, ea16s):
        for g in range(CH // 16):
            for k in range(16):
                e = g * 16 + k
                eav = jnp.full((16,), ea16s[g][k], jnp.float32)
                for v in range(F // 16):
                    sl = pl.ds(v * 16, 16)
                    rows[b, e, sl] = jnp.maximum(
                        rows[b, e, sl] + eav * wc[v], 0.0)

    _stage(0, 0)
    _stage_wait(0)
    _stage(1, 1)
    _unpack(0, 0)
    _gather(0)

    def _pair(p, c):
        for b in range(2):
            j = 2 * p + b
            nb = 1 - b

            @pl.when(j + 1 < NCHUNK)
            def _prep_next():
                sbp = lax.rem(lax.div(j + 1, BPC), 2)

                @pl.when(jnp.logical_and(lax.rem(j + 1, BPC) == 0, sbp == 0))
                def _():
                    _stage_wait(0)

                @pl.when(jnp.logical_and(lax.rem(j + 1, BPC) == 0, sbp == 1))
                def _():
                    _stage_wait(1)

                @pl.when(j >= 1)
                def _():
                    _scatter_wait(nb)
                _unpack(j + 1, nb)
                _gather(nb)

            jsb = lax.rem(lax.div(j, BPC), 2)
            joff = lax.rem(j, BPC) * CH
            ea16s = [eab[jsb, pl.ds(joff + g * 16, 16)]
                     for g in range(CH // 16)]

            nblk = lax.div(j + 1, BPC) + 1
            adv = jnp.logical_and(lax.rem(j + 1, BPC) == 0, nblk < NBLK)

            @pl.when(jnp.logical_and(adv, lax.rem(nblk, 2) == 0))
            def _stage_ahead0():
                _stage(nblk, 0)

            @pl.when(jnp.logical_and(adv, lax.rem(nblk, 2) == 1))
            def _stage_ahead1():
                _stage(nblk, 1)
            _gather_wait(b)
            _compute(b, ea16s)
            _scatter(b)
        return c
    lax.fori_loop(0, NCHUNK // 2, _pair, 0)

    _scatter_wait(0)
    _scatter_wait(1)

    plsc.subcore_barrier()

    pltpu.sync_copy(acc_sh.at[pl.ds(sid * RPT, RPT)],
                    acc_out.at[pl.ds(sid * RPT, RPT), pl.ds(cid * F, F)])
    if with_count:
        @pl.when(jnp.logical_and(cid == 0, sid == 0))
        def _copy_cnt():
            pltpu.sync_copy(cnt_sh, cnt_out)


def _make_sc_layer(with_count):
    mesh = plsc.VectorSubcoreMesh(
        core_axis_name="c", subcore_axis_name="s",
        num_cores=NC, num_subcores=NS)
    out_type = [jax.ShapeDtypeStruct((NP, D), jnp.float32)]
    scratch = [
        pltpu.VMEM((2, SBLK), jnp.int32),
        pltpu.VMEM((2, SBLK), jnp.int32),
        pltpu.VMEM((2, SBLK), jnp.float32),
        pltpu.VMEM((2, CH), jnp.int32),
        pltpu.VMEM((2, CH), jnp.int32),
        pltpu.VMEM((2, CH, F), jnp.float32),
        pltpu.VMEM((F,), jnp.float32),
        pltpu.VMEM((ZR, F), jnp.float32),
    ]
    if with_count:
        out_type.append(jax.ShapeDtypeStruct((NP,), jnp.float32))
        scratch += [
            pltpu.VMEM((ZF,), jnp.float32),
            pltpu.VMEM((CH,), jnp.float32),
        ]
    scratch += [pltpu.VMEM_SHARED((NP, F), jnp.float32)]
    if with_count:
        scratch += [pltpu.VMEM_SHARED((NP,), jnp.float32)]
    nsem = 8 if with_count else 6
    scratch += [pltpu.SemaphoreType.DMA] * nsem
    return pl.kernel(
        functools.partial(_sc_layer_body, with_count),
        out_type=tuple(out_type),
        mesh=mesh,
        scratch_types=scratch,
        compiler_params=_SC_PARAMS,
    )


_sc_layer_with_count = _make_sc_layer(True)
_sc_layer_no_count = _make_sc_layer(False)


def kernel(x, edge_index, edge_attr, W1, b1, W2, b2):
    ei = edge_index.astype(jnp.int32)
    ea = edge_attr.astype(jnp.float32)

    w1k, wcol1 = W1[:, :D], W1[:, D].reshape(NC, F)
    w2k, wcol2 = W2[:, :D], W2[:, D].reshape(NC, F)
    b1r = b1.reshape(NC, 1, F)
    b2r = b2.reshape(NC, 1, F)

    y1, src, dst = _linear(x.astype(jnp.float32), w1k, b1r, ei)
    acc1, cnt = _sc_layer_with_count(y1, src, dst, ea, wcol1)
    y2, r = _combine_linear(acc1, cnt.reshape(NP, 1), w2k, b2r)
    acc2, = _sc_layer_no_count(y2, src, dst, ea, wcol2)
    return _finalize(acc2, r)
